# baseline (speedup 1.0000x reference)
import functools

import jax
import jax.numpy as jnp
from jax import lax
from jax.experimental import pallas as pl
from jax.experimental.pallas import tpu as pltpu
from jax.experimental.pallas import tpu_sc as plsc

VOCAB = 352899
EMB = 50
EMBP = 128
OUT = 1024
TOKENS = 26 * 4096

NC = 2
NS = 16
NW = NC * NS

B_PER_W = TOKENS // NW
CHUNK = 128
NCH = B_PER_W // CHUNK
NBUF = 2

PAD_RB = 2048


def _pad_body(e_ref, o_ref):
    o_ref[...] = jnp.concatenate(
        [e_ref[...], jnp.zeros((PAD_RB, EMBP - EMB), jnp.float32)], axis=1
    )


def _tc_pad_table(emb):
    grid = (pl.cdiv(VOCAB, PAD_RB),)
    return pl.pallas_call(
        _pad_body,
        grid=grid,
        in_specs=[pl.BlockSpec((PAD_RB, EMB), lambda i: (i, 0))],
        out_specs=pl.BlockSpec((PAD_RB, EMBP), lambda i: (i, 0)),
        out_shape=jax.ShapeDtypeStruct((VOCAB, EMBP), jnp.float32),
        compiler_params=pltpu.CompilerParams(
            dimension_semantics=("arbitrary",),
        ),
    )(emb)


def _make_sc_gather():
    mesh = plsc.VectorSubcoreMesh(core_axis_name="c", subcore_axis_name="s")

    @functools.partial(
        pl.kernel,
        mesh=mesh,
        out_type=jax.ShapeDtypeStruct((TOKENS, EMBP), jnp.float32),
        scratch_types=[
            pltpu.VMEM((NCH, CHUNK), jnp.int32),
            pltpu.VMEM((NBUF, CHUNK, EMBP), jnp.float32),
            [pltpu.SemaphoreType.DMA] * NBUF,
        ],
    )
    def gather_kernel(table_hbm, idx_hbm, out_hbm, idx_v, rows_v, sems):
        wid = lax.axis_index("s") * NC + lax.axis_index("c")
        base = wid * B_PER_W
        pltpu.sync_copy(idx_hbm.at[wid], idx_v)

        def start(j, b):
            pltpu.async_copy(table_hbm.at[idx_v.at[j]], rows_v.at[b], sems[b])

        def drain(j, b):
            pltpu.make_async_copy(
                table_hbm.at[idx_v.at[j]], rows_v.at[b], sems[b]
            ).wait()
            pltpu.sync_copy(rows_v.at[b], out_hbm.at[pl.ds(base + j * CHUNK, CHUNK)])

        start(0, 0)

        def body(i, _):
            j = i * NBUF
            start(j + 1, 1)
            drain(j, 0)
            @pl.when(j + 2 < NCH)
            def _():
                start(j + 2, 0)
            drain(j + 1, 1)
            return ()

        lax.fori_loop(0, NCH // NBUF, body, ())

    return gather_kernel


_sc_gather = _make_sc_gather()


BT = 1024


def _matmul_body(v_ref, wt_ref, b_ref, o_ref):
    o_ref[...] = (
        jnp.dot(v_ref[...], wt_ref[...], preferred_element_type=jnp.float32)
        + b_ref[...]
    )


def _tc_project(v, wt, b2d):
    grid = (TOKENS // BT,)
    return pl.pallas_call(
        _matmul_body,
        grid=grid,
        in_specs=[
            pl.BlockSpec((BT, EMBP), lambda i: (i, 0)),
            pl.BlockSpec((EMBP, OUT), lambda i: (0, 0)),
            pl.BlockSpec((1, OUT), lambda i: (0, 0)),
        ],
        out_specs=pl.BlockSpec((BT, OUT), lambda i: (i, 0)),
        out_shape=jax.ShapeDtypeStruct((TOKENS, OUT), jnp.float32),
        compiler_params=pltpu.CompilerParams(
            dimension_semantics=("arbitrary",),
        ),
    )(v, wt, b2d)


def kernel(x, emb, W, b):
    idx = jnp.transpose(x, (1, 0)).reshape(NW, NCH, CHUNK).astype(jnp.int32)
    emb_p = _tc_pad_table(emb)
    v = _sc_gather(emb_p, idx)
    wt_p = jnp.pad(W.T, ((0, EMBP - EMB), (0, 0)))
    y = _tc_project(v, wt_p, b.reshape(1, OUT))
    return y.reshape(4096, 26, OUT)


# speedup vs baseline: 1.3111x; 1.3111x over previous
import functools

import jax
import jax.numpy as jnp
from jax import lax
from jax.experimental import pallas as pl
from jax.experimental.pallas import tpu as pltpu
from jax.experimental.pallas import tpu_sc as plsc

VOCAB = 352899
EMB = 50
EMBP = 128
OUT = 1024
NPOS = 4096
NFEAT = 26
TOKENS = NPOS * NFEAT

NC = 2
NS = 16
NW = NC * NS

T_PER_W = NPOS // NW
TPC = 4
CPI = TPC * NFEAT
NCH = T_PER_W // TPC
NBUF = 2

PAD_RB = 2048


def _pad_body(e_ref, o_ref):
    o_ref[...] = jnp.concatenate(
        [e_ref[...], jnp.zeros((PAD_RB, EMBP - EMB), jnp.float32)], axis=1
    )


def _tc_pad_table(emb):
    grid = (pl.cdiv(VOCAB, PAD_RB),)
    return pl.pallas_call(
        _pad_body,
        grid=grid,
        in_specs=[pl.BlockSpec((PAD_RB, EMB), lambda i: (i, 0))],
        out_specs=pl.BlockSpec((PAD_RB, EMBP), lambda i: (i, 0)),
        out_shape=jax.ShapeDtypeStruct((VOCAB, EMBP), jnp.float32),
        compiler_params=pltpu.CompilerParams(
            dimension_semantics=("arbitrary",),
        ),
    )(emb)


def _make_sc_gather():
    mesh = plsc.VectorSubcoreMesh(core_axis_name="c", subcore_axis_name="s")

    @functools.partial(
        pl.kernel,
        mesh=mesh,
        out_type=jax.ShapeDtypeStruct((NPOS, NFEAT, EMBP), jnp.float32),
        scratch_types=[
            pltpu.VMEM((NCH, CPI), jnp.int32),
            pltpu.VMEM((NBUF, CPI, EMBP), jnp.float32),
            [pltpu.SemaphoreType.DMA] * NBUF,
        ],
    )
    def gather_kernel(table_hbm, idx_hbm, out_hbm, idx_v, rows_v, sems):
        wid = lax.axis_index("s") * NC + lax.axis_index("c")
        base = wid * T_PER_W
        pltpu.sync_copy(idx_hbm.at[wid], idx_v)

        def start(j, b):
            pltpu.async_copy(table_hbm.at[idx_v.at[j]], rows_v.at[b], sems[b])

        def drain(j, b):
            pltpu.make_async_copy(
                table_hbm.at[idx_v.at[j]], rows_v.at[b], sems[b]
            ).wait()
            for k in range(TPC):
                pltpu.sync_copy(
                    rows_v.at[b].at[pl.ds(k * NFEAT, NFEAT)],
                    out_hbm.at[base + j * TPC + k],
                )

        start(0, 0)

        def body(i, _):
            j = i * NBUF
            start(j + 1, 1)
            drain(j, 0)
            @pl.when(j + 2 < NCH)
            def _():
                start(j + 2, 0)
            drain(j + 1, 1)
            return ()

        lax.fori_loop(0, NCH // NBUF, body, ())

    return gather_kernel


_sc_gather = _make_sc_gather()


TB = 32


def _matmul_body(v_ref, wt_ref, b_ref, o_ref):
    for t in range(TB):
        o_ref[t] = (
            jnp.dot(v_ref[t], wt_ref[...], preferred_element_type=jnp.float32)
            + b_ref[...]
        )


def _tc_project(v3, wt, b2d):
    grid = (NPOS // TB,)
    return pl.pallas_call(
        _matmul_body,
        grid=grid,
        in_specs=[
            pl.BlockSpec((TB, NFEAT, EMBP), lambda i: (i, 0, 0)),
            pl.BlockSpec((EMBP, OUT), lambda i: (0, 0)),
            pl.BlockSpec((1, OUT), lambda i: (0, 0)),
        ],
        out_specs=pl.BlockSpec((TB, NFEAT, OUT), lambda i: (i, 0, 0)),
        out_shape=jax.ShapeDtypeStruct((NPOS, NFEAT, OUT), jnp.float32),
        compiler_params=pltpu.CompilerParams(
            dimension_semantics=("arbitrary",),
        ),
    )(v3, wt, b2d)


def kernel(x, emb, W, b):
    idx = jnp.transpose(x, (1, 0)).reshape(NW, NCH, CPI).astype(jnp.int32)
    emb_p = _tc_pad_table(emb)
    v3 = _sc_gather(emb_p, idx)
    wt_p = jnp.pad(W.T, ((0, EMBP - EMB), (0, 0)))
    return _tc_project(v3, wt_p, b.reshape(1, OUT))


# speedup vs baseline: 3.2918x; 2.5107x over previous
import functools

import jax
import jax.numpy as jnp
from jax import lax
from jax.experimental import pallas as pl
from jax.experimental.pallas import tpu as pltpu
from jax.experimental.pallas import tpu_sc as plsc

VOCAB = 352899
EMB = 50
EMBP = 128
OUT = 1024
NPOS = 4096
NFEAT = 26

NC = 2
NS = 16
NW = NC * NS

T_PER_W = NPOS // NW
CHUNK = 128
NBUF = 2

WID_CB = 2048


def _widen_body(et_ref, o_ref):
    o_ref[...] = jnp.concatenate(
        [
            jnp.transpose(et_ref[...], (1, 0)),
            jnp.zeros((WID_CB, EMBP - EMB), jnp.float32),
        ],
        axis=1,
    )


def _tc_widen_table(emb_t):
    grid = (pl.cdiv(VOCAB, WID_CB),)
    return pl.pallas_call(
        _widen_body,
        grid=grid,
        in_specs=[pl.BlockSpec((EMB, WID_CB), lambda i: (0, i))],
        out_specs=pl.BlockSpec((WID_CB, EMBP), lambda i: (i, 0)),
        out_shape=jax.ShapeDtypeStruct((VOCAB, EMBP), jnp.float32),
        compiler_params=pltpu.CompilerParams(
            dimension_semantics=("arbitrary",),
        ),
    )(emb_t)


def _make_sc_gather():
    mesh = plsc.VectorSubcoreMesh(core_axis_name="c", subcore_axis_name="s")

    @functools.partial(
        pl.kernel,
        mesh=mesh,
        out_type=jax.ShapeDtypeStruct((NFEAT, NPOS, EMBP), jnp.float32),
        scratch_types=[
            pltpu.VMEM((NFEAT, CHUNK), jnp.int32),
            pltpu.VMEM((NBUF, CHUNK, EMBP), jnp.float32),
            [pltpu.SemaphoreType.DMA] * NBUF,
        ],
    )
    def gather_kernel(table_hbm, x_hbm, out_hbm, idx_v, rows_v, sems):
        wid = lax.axis_index("s") * NC + lax.axis_index("c")
        base = wid * T_PER_W
        pltpu.sync_copy(x_hbm.at[:, pl.ds(base, T_PER_W)], idx_v)

        def start(j, b):
            pltpu.async_copy(table_hbm.at[idx_v.at[j]], rows_v.at[b], sems[b])

        def drain(j, b):
            pltpu.make_async_copy(
                table_hbm.at[idx_v.at[j]], rows_v.at[b], sems[b]
            ).wait()
            pltpu.sync_copy(rows_v.at[b], out_hbm.at[j].at[pl.ds(base, T_PER_W)])

        start(0, 0)

        def body(i, _):
            j = i * NBUF
            start(j + 1, 1)
            drain(j, 0)
            @pl.when(j + 2 < NFEAT)
            def _():
                start(j + 2, 0)
            drain(j + 1, 1)
            return ()

        lax.fori_loop(0, NFEAT // NBUF, body, ())

    return gather_kernel


_sc_gather = _make_sc_gather()


TB = 64


def _matmul_body(v_ref, wt_ref, b_ref, o_ref):
    for c in range(NFEAT):
        o_ref[c] = (
            jnp.dot(v_ref[c], wt_ref[...], preferred_element_type=jnp.float32)
            + b_ref[...]
        )


def _tc_project(v_perm, wt, b2d):
    grid = (NPOS // TB,)
    return pl.pallas_call(
        _matmul_body,
        grid=grid,
        in_specs=[
            pl.BlockSpec((NFEAT, TB, EMBP), lambda i: (0, i, 0)),
            pl.BlockSpec((EMBP, OUT), lambda i: (0, 0)),
            pl.BlockSpec((1, OUT), lambda i: (0, 0)),
        ],
        out_specs=pl.BlockSpec((NFEAT, TB, OUT), lambda i: (0, i, 0)),
        out_shape=jax.ShapeDtypeStruct((NFEAT, NPOS, OUT), jnp.float32),
        compiler_params=pltpu.CompilerParams(
            dimension_semantics=("arbitrary",),
        ),
    )(v_perm, wt, b2d)


def kernel(x, emb, W, b):
    emb_t = jnp.transpose(emb, (1, 0))
    table = _tc_widen_table(emb_t)
    v_perm = _sc_gather(table, x)
    wt_p = jnp.pad(jnp.transpose(W, (1, 0)), ((0, EMBP - EMB), (0, 0)))
    y_perm = _tc_project(v_perm, wt_p, b.reshape(1, OUT))
    return jnp.transpose(y_perm, (1, 0, 2))


# speedup vs baseline: 3.7881x; 1.1508x over previous
import functools

import jax
import jax.numpy as jnp
from jax import lax
from jax.experimental import pallas as pl
from jax.experimental.pallas import tpu as pltpu
from jax.experimental.pallas import tpu_sc as plsc

VOCAB = 352899
EMB = 50
EMBP = 128
OUT = 1024
NPOS = 4096
NFEAT = 26

NC = 2
NS = 16
NW = NC * NS

T_PER_W = NPOS // NW
CHUNK = 128
NBUF = 2

WID_CB = 4096


def _widen_body(et_ref, o_ref):
    o_ref[...] = jnp.concatenate(
        [
            jnp.transpose(et_ref[...], (1, 0)),
            jnp.zeros((WID_CB, EMBP - EMB), jnp.float32),
        ],
        axis=1,
    )


def _tc_widen_table(emb_t):
    grid = (pl.cdiv(VOCAB, WID_CB),)
    return pl.pallas_call(
        _widen_body,
        grid=grid,
        in_specs=[pl.BlockSpec((EMB, WID_CB), lambda i: (0, i))],
        out_specs=pl.BlockSpec((WID_CB, EMBP), lambda i: (i, 0)),
        out_shape=jax.ShapeDtypeStruct((VOCAB, EMBP), jnp.float32),
        compiler_params=pltpu.CompilerParams(
            dimension_semantics=("arbitrary",),
        ),
    )(emb_t)


def _make_sc_gather():
    mesh = plsc.VectorSubcoreMesh(core_axis_name="c", subcore_axis_name="s")

    @functools.partial(
        pl.kernel,
        mesh=mesh,
        out_type=jax.ShapeDtypeStruct((NFEAT, NPOS, EMBP), jnp.float32),
        scratch_types=[
            pltpu.VMEM((NFEAT, CHUNK), jnp.int32),
            pltpu.VMEM((NBUF, CHUNK, EMBP), jnp.float32),
            [pltpu.SemaphoreType.DMA] * NBUF,
        ],
    )
    def gather_kernel(table_hbm, x_hbm, out_hbm, idx_v, rows_v, sems):
        wid = lax.axis_index("s") * NC + lax.axis_index("c")
        base = wid * T_PER_W
        pltpu.sync_copy(x_hbm.at[:, pl.ds(base, T_PER_W)], idx_v)

        def start(j, b):
            pltpu.async_copy(table_hbm.at[idx_v.at[j]], rows_v.at[b], sems[b])

        def drain(j, b):
            pltpu.make_async_copy(
                table_hbm.at[idx_v.at[j]], rows_v.at[b], sems[b]
            ).wait()
            pltpu.sync_copy(rows_v.at[b], out_hbm.at[j].at[pl.ds(base, T_PER_W)])

        start(0, 0)

        def body(i, _):
            j = i * NBUF
            start(j + 1, 1)
            drain(j, 0)
            @pl.when(j + 2 < NFEAT)
            def _():
                start(j + 2, 0)
            drain(j + 1, 1)
            return ()

        lax.fori_loop(0, NFEAT // NBUF, body, ())

    return gather_kernel


_sc_gather = _make_sc_gather()


TB = 128


def _matmul_body(v_ref, wt_ref, b_ref, o_ref):
    for c in range(NFEAT):
        o_ref[c] = (
            jnp.dot(v_ref[c], wt_ref[...], preferred_element_type=jnp.float32)
            + b_ref[...]
        )


def _tc_project(v_perm, wt, b2d):
    grid = (NPOS // TB,)
    return pl.pallas_call(
        _matmul_body,
        grid=grid,
        in_specs=[
            pl.BlockSpec((NFEAT, TB, EMBP), lambda i: (0, i, 0)),
            pl.BlockSpec((EMBP, OUT), lambda i: (0, 0)),
            pl.BlockSpec((1, OUT), lambda i: (0, 0)),
        ],
        out_specs=pl.BlockSpec((NFEAT, TB, OUT), lambda i: (0, i, 0)),
        out_shape=jax.ShapeDtypeStruct((NFEAT, NPOS, OUT), jnp.float32),
        compiler_params=pltpu.CompilerParams(
            dimension_semantics=("arbitrary",),
        ),
    )(v_perm, wt, b2d)


def kernel(x, emb, W, b):
    emb_t = jnp.transpose(emb, (1, 0))
    table = _tc_widen_table(emb_t)
    v_perm = _sc_gather(table, x)
    wt_p = jnp.pad(jnp.transpose(W, (1, 0)), ((0, EMBP - EMB), (0, 0)))
    y_perm = _tc_project(v_perm, wt_p, b.reshape(1, OUT))
    return jnp.transpose(y_perm, (1, 0, 2))


# speedup vs baseline: 4.0964x; 1.0814x over previous
import functools

import jax
import jax.numpy as jnp
from jax import lax
from jax.experimental import pallas as pl
from jax.experimental.pallas import tpu as pltpu
from jax.experimental.pallas import tpu_sc as plsc

VOCAB = 352899
EMB = 50
EMBP = 128
OUT = 1024
NPOS = 4096
NFEAT = 26

NC = 2
NS = 16
NW = NC * NS

T_PER_W = NPOS // NW
CHUNK = 128
NBUF = 2

WID_CB = 8192


def _widen_body(et_ref, o_ref):
    o_ref[...] = jnp.concatenate(
        [
            jnp.transpose(et_ref[...], (1, 0)),
            jnp.zeros((WID_CB, EMBP - EMB), jnp.float32),
        ],
        axis=1,
    )


def _tc_widen_table(emb_t):
    grid = (pl.cdiv(VOCAB, WID_CB),)
    return pl.pallas_call(
        _widen_body,
        grid=grid,
        in_specs=[pl.BlockSpec((EMB, WID_CB), lambda i: (0, i))],
        out_specs=pl.BlockSpec((WID_CB, EMBP), lambda i: (i, 0)),
        out_shape=jax.ShapeDtypeStruct((VOCAB, EMBP), jnp.float32),
        compiler_params=pltpu.CompilerParams(
            dimension_semantics=("arbitrary",),
        ),
    )(emb_t)


def _make_sc_gather():
    mesh = plsc.VectorSubcoreMesh(core_axis_name="c", subcore_axis_name="s")

    @functools.partial(
        pl.kernel,
        mesh=mesh,
        out_type=jax.ShapeDtypeStruct((NFEAT, NPOS, EMBP), jnp.float32),
        scratch_types=[
            pltpu.VMEM((NFEAT, CHUNK), jnp.int32),
            pltpu.VMEM((NBUF, CHUNK, EMBP), jnp.float32),
            [pltpu.SemaphoreType.DMA] * NBUF,
        ],
    )
    def gather_kernel(table_hbm, x_hbm, out_hbm, idx_v, rows_v, sems):
        wid = lax.axis_index("s") * NC + lax.axis_index("c")
        base = wid * T_PER_W
        pltpu.sync_copy(x_hbm.at[:, pl.ds(base, T_PER_W)], idx_v)

        def start(j, b):
            pltpu.async_copy(table_hbm.at[idx_v.at[j]], rows_v.at[b], sems[b])

        def drain(j, b):
            pltpu.make_async_copy(
                table_hbm.at[idx_v.at[j]], rows_v.at[b], sems[b]
            ).wait()
            pltpu.sync_copy(rows_v.at[b], out_hbm.at[j].at[pl.ds(base, T_PER_W)])

        start(0, 0)

        def body(i, _):
            j = i * NBUF
            start(j + 1, 1)
            drain(j, 0)
            @pl.when(j + 2 < NFEAT)
            def _():
                start(j + 2, 0)
            drain(j + 1, 1)
            return ()

        lax.fori_loop(0, NFEAT // NBUF, body, ())

    return gather_kernel


_sc_gather = _make_sc_gather()


TB = 128


def _matmul_body(v_ref, wt_ref, b_ref, o_ref):
    for c in range(NFEAT):
        o_ref[c] = (
            jnp.dot(v_ref[c], wt_ref[...], preferred_element_type=jnp.float32)
            + b_ref[...]
        )


def _tc_project(v_perm, wt, b2d):
    grid = (NPOS // TB,)
    return pl.pallas_call(
        _matmul_body,
        grid=grid,
        in_specs=[
            pl.BlockSpec((NFEAT, TB, EMBP), lambda i: (0, i, 0)),
            pl.BlockSpec((EMBP, OUT), lambda i: (0, 0)),
            pl.BlockSpec((1, OUT), lambda i: (0, 0)),
        ],
        out_specs=pl.BlockSpec((NFEAT, TB, OUT), lambda i: (0, i, 0)),
        out_shape=jax.ShapeDtypeStruct((NFEAT, NPOS, OUT), jnp.float32),
        compiler_params=pltpu.CompilerParams(
            dimension_semantics=("arbitrary",),
        ),
    )(v_perm, wt, b2d)


def kernel(x, emb, W, b):
    emb_t = jnp.transpose(emb, (1, 0))
    table = _tc_widen_table(emb_t)
    v_perm = _sc_gather(table, x)
    wt_p = jnp.pad(jnp.transpose(W, (1, 0)), ((0, EMBP - EMB), (0, 0)))
    y_perm = _tc_project(v_perm, wt_p, b.reshape(1, OUT))
    return jnp.transpose(y_perm, (1, 0, 2))


# speedup vs baseline: 4.2323x; 1.0332x over previous
import functools

import jax
import jax.numpy as jnp
from jax import lax
from jax.experimental import pallas as pl
from jax.experimental.pallas import tpu as pltpu
from jax.experimental.pallas import tpu_sc as plsc

VOCAB = 352899
EMB = 50
EMBP = 128
OUT = 1024
NPOS = 4096
NFEAT = 26

NC = 2
NS = 16
NW = NC * NS

T_PER_W = NPOS // NW
CHUNK = 128
NBUF = 2

WID_CB = 16384


def _widen_body(et_ref, o_ref):
    o_ref[...] = jnp.concatenate(
        [
            jnp.transpose(et_ref[...], (1, 0)),
            jnp.zeros((WID_CB, EMBP - EMB), jnp.float32),
        ],
        axis=1,
    )


def _tc_widen_table(emb_t):
    grid = (pl.cdiv(VOCAB, WID_CB),)
    return pl.pallas_call(
        _widen_body,
        grid=grid,
        in_specs=[pl.BlockSpec((EMB, WID_CB), lambda i: (0, i))],
        out_specs=pl.BlockSpec((WID_CB, EMBP), lambda i: (i, 0)),
        out_shape=jax.ShapeDtypeStruct((VOCAB, EMBP), jnp.float32),
        compiler_params=pltpu.CompilerParams(
            dimension_semantics=("arbitrary",),
        ),
    )(emb_t)


def _make_sc_gather():
    mesh = plsc.VectorSubcoreMesh(core_axis_name="c", subcore_axis_name="s")

    @functools.partial(
        pl.kernel,
        mesh=mesh,
        out_type=jax.ShapeDtypeStruct((NFEAT, NPOS, EMBP), jnp.float32),
        scratch_types=[
            pltpu.VMEM((NFEAT, CHUNK), jnp.int32),
            pltpu.VMEM((NBUF, CHUNK, EMBP), jnp.float32),
            [pltpu.SemaphoreType.DMA] * NBUF,
        ],
    )
    def gather_kernel(table_hbm, x_hbm, out_hbm, idx_v, rows_v, sems):
        wid = lax.axis_index("s") * NC + lax.axis_index("c")
        base = wid * T_PER_W
        pltpu.sync_copy(x_hbm.at[:, pl.ds(base, T_PER_W)], idx_v)

        def start(j, b):
            pltpu.async_copy(table_hbm.at[idx_v.at[j]], rows_v.at[b], sems[b])

        def drain(j, b):
            pltpu.make_async_copy(
                table_hbm.at[idx_v.at[j]], rows_v.at[b], sems[b]
            ).wait()
            pltpu.sync_copy(rows_v.at[b], out_hbm.at[j].at[pl.ds(base, T_PER_W)])

        start(0, 0)

        def body(i, _):
            j = i * NBUF
            start(j + 1, 1)
            drain(j, 0)
            @pl.when(j + 2 < NFEAT)
            def _():
                start(j + 2, 0)
            drain(j + 1, 1)
            return ()

        lax.fori_loop(0, NFEAT // NBUF, body, ())

    return gather_kernel


_sc_gather = _make_sc_gather()


TB = 128


def _matmul_body(v_ref, wt_ref, b_ref, o_ref):
    for c in range(NFEAT):
        o_ref[c] = (
            jnp.dot(v_ref[c], wt_ref[...], preferred_element_type=jnp.float32)
            + b_ref[...]
        )


def _tc_project(v_perm, wt, b2d):
    grid = (NPOS // TB,)
    return pl.pallas_call(
        _matmul_body,
        grid=grid,
        in_specs=[
            pl.BlockSpec((NFEAT, TB, EMBP), lambda i: (0, i, 0)),
            pl.BlockSpec((EMBP, OUT), lambda i: (0, 0)),
            pl.BlockSpec((1, OUT), lambda i: (0, 0)),
        ],
        out_specs=pl.BlockSpec((NFEAT, TB, OUT), lambda i: (0, i, 0)),
        out_shape=jax.ShapeDtypeStruct((NFEAT, NPOS, OUT), jnp.float32),
        compiler_params=pltpu.CompilerParams(
            dimension_semantics=("arbitrary",),
        ),
    )(v_perm, wt, b2d)


def kernel(x, emb, W, b):
    emb_t = jnp.transpose(emb, (1, 0))
    table = _tc_widen_table(emb_t)
    v_perm = _sc_gather(table, x)
    wt_p = jnp.pad(jnp.transpose(W, (1, 0)), ((0, EMBP - EMB), (0, 0)))
    y_perm = _tc_project(v_perm, wt_p, b.reshape(1, OUT))
    return jnp.transpose(y_perm, (1, 0, 2))


# speedup vs baseline: 4.2605x; 1.0067x over previous
import functools

import jax
import jax.numpy as jnp
from jax import lax
from jax.experimental import pallas as pl
from jax.experimental.pallas import tpu as pltpu
from jax.experimental.pallas import tpu_sc as plsc

VOCAB = 352899
EMB = 50
EMBP = 128
OUT = 1024
NPOS = 4096
NFEAT = 26

NC = 2
NS = 16
NW = NC * NS

T_PER_W = NPOS // NW
CHUNK = 128
NBUF = 2

WID_CB = 32768


def _widen_body(et_ref, o_ref):
    o_ref[...] = jnp.concatenate(
        [
            jnp.transpose(et_ref[...], (1, 0)),
            jnp.zeros((WID_CB, EMBP - EMB), jnp.float32),
        ],
        axis=1,
    )


def _tc_widen_table(emb_t):
    grid = (pl.cdiv(VOCAB, WID_CB),)
    return pl.pallas_call(
        _widen_body,
        grid=grid,
        in_specs=[pl.BlockSpec((EMB, WID_CB), lambda i: (0, i))],
        out_specs=pl.BlockSpec((WID_CB, EMBP), lambda i: (i, 0)),
        out_shape=jax.ShapeDtypeStruct((VOCAB, EMBP), jnp.float32),
        compiler_params=pltpu.CompilerParams(
            dimension_semantics=("arbitrary",),
        ),
    )(emb_t)


def _make_sc_gather():
    mesh = plsc.VectorSubcoreMesh(core_axis_name="c", subcore_axis_name="s")

    @functools.partial(
        pl.kernel,
        mesh=mesh,
        out_type=jax.ShapeDtypeStruct((NFEAT, NPOS, EMBP), jnp.float32),
        scratch_types=[
            pltpu.VMEM((NFEAT, CHUNK), jnp.int32),
            pltpu.VMEM((NBUF, CHUNK, EMBP), jnp.float32),
            [pltpu.SemaphoreType.DMA] * NBUF,
        ],
    )
    def gather_kernel(table_hbm, x_hbm, out_hbm, idx_v, rows_v, sems):
        wid = lax.axis_index("s") * NC + lax.axis_index("c")
        base = wid * T_PER_W
        pltpu.sync_copy(x_hbm.at[:, pl.ds(base, T_PER_W)], idx_v)

        def start(j, b):
            pltpu.async_copy(table_hbm.at[idx_v.at[j]], rows_v.at[b], sems[b])

        def drain(j, b):
            pltpu.make_async_copy(
                table_hbm.at[idx_v.at[j]], rows_v.at[b], sems[b]
            ).wait()
            pltpu.sync_copy(rows_v.at[b], out_hbm.at[j].at[pl.ds(base, T_PER_W)])

        start(0, 0)

        def body(i, _):
            j = i * NBUF
            start(j + 1, 1)
            drain(j, 0)
            @pl.when(j + 2 < NFEAT)
            def _():
                start(j + 2, 0)
            drain(j + 1, 1)
            return ()

        lax.fori_loop(0, NFEAT // NBUF, body, ())

    return gather_kernel


_sc_gather = _make_sc_gather()


TB = 128


def _matmul_body(v_ref, wt_ref, b_ref, o_ref):
    for c in range(NFEAT):
        o_ref[c] = (
            jnp.dot(v_ref[c], wt_ref[...], preferred_element_type=jnp.float32)
            + b_ref[...]
        )


def _tc_project(v_perm, wt, b2d):
    grid = (NPOS // TB,)
    return pl.pallas_call(
        _matmul_body,
        grid=grid,
        in_specs=[
            pl.BlockSpec((NFEAT, TB, EMBP), lambda i: (0, i, 0)),
            pl.BlockSpec((EMBP, OUT), lambda i: (0, 0)),
            pl.BlockSpec((1, OUT), lambda i: (0, 0)),
        ],
        out_specs=pl.BlockSpec((NFEAT, TB, OUT), lambda i: (0, i, 0)),
        out_shape=jax.ShapeDtypeStruct((NFEAT, NPOS, OUT), jnp.float32),
        compiler_params=pltpu.CompilerParams(
            dimension_semantics=("arbitrary",),
        ),
    )(v_perm, wt, b2d)


def kernel(x, emb, W, b):
    emb_t = jnp.transpose(emb, (1, 0))
    table = _tc_widen_table(emb_t)
    v_perm = _sc_gather(table, x)
    wt_p = jnp.pad(jnp.transpose(W, (1, 0)), ((0, EMBP - EMB), (0, 0)))
    y_perm = _tc_project(v_perm, wt_p, b.reshape(1, OUT))
    return jnp.transpose(y_perm, (1, 0, 2))
